# baseline (speedup 1.0000x reference)
import functools

import jax
import jax.numpy as jnp
from jax import lax
from jax.experimental import pallas as pl
from jax.experimental.pallas import tpu as pltpu
from jax.experimental.pallas import tpu_sc as plsc

VOCAB = 1000000
EMBED_DIM = 64
BATCH = 16384

NUM_CORES = 2
NUM_SUBCORES = 16
NUM_WORKERS = NUM_CORES * NUM_SUBCORES
B_PER_W = BATCH // NUM_WORKERS
CHUNK = 128
N_CHUNKS = B_PER_W // CHUNK

_mesh = plsc.VectorSubcoreMesh(core_axis_name="c", subcore_axis_name="s")


@functools.partial(
    pl.kernel,
    mesh=_mesh,
    out_type=jax.ShapeDtypeStruct((BATCH, EMBED_DIM), jnp.float32),
    scratch_types=[
        pltpu.VMEM((N_CHUNKS, CHUNK), jnp.int32),
        pltpu.VMEM((B_PER_W, EMBED_DIM), jnp.float32),
        pltpu.SemaphoreType.DMA,
    ],
    compiler_params=pltpu.CompilerParams(use_tc_tiling_on_sc=False),
)
def _gather_kernel(table_hbm, idx_hbm, out_hbm, idx_v, rows_v, sem):
    wid = lax.axis_index("s") * NUM_CORES + lax.axis_index("c")
    base = wid * B_PER_W
    pltpu.sync_copy(idx_hbm.at[pl.ds(wid * N_CHUNKS, N_CHUNKS)], idx_v)
    copies = [
        pltpu.async_copy(
            table_hbm.at[idx_v.at[j]],
            rows_v.at[pl.ds(j * CHUNK, CHUNK)],
            sem,
        )
        for j in range(N_CHUNKS)
    ]
    for cp in copies:
        cp.wait()
    pltpu.sync_copy(rows_v, out_hbm.at[pl.ds(base, B_PER_W)])


def kernel(embed_cache, indices):
    idx = indices.astype(jnp.int32).reshape(NUM_WORKERS * N_CHUNKS, CHUNK)
    return _gather_kernel(embed_cache, idx)


# speedup vs baseline: 2.1650x; 2.1650x over previous
import functools

import jax
import jax.numpy as jnp
from jax import lax
from jax.experimental import pallas as pl
from jax.experimental.pallas import tpu as pltpu
from jax.experimental.pallas import tpu_sc as plsc

VOCAB = 1000000
EMBED_DIM = 64
BATCH = 16384

NUM_CORES = 2
NUM_SUBCORES = 16
NUM_WORKERS = NUM_CORES * NUM_SUBCORES
B_PER_W = BATCH // NUM_WORKERS
CHUNK = 32
N_CHUNKS = B_PER_W // CHUNK
GROUP = 8
LANES = 16

_mesh = plsc.VectorSubcoreMesh(core_axis_name="c", subcore_axis_name="s")


@functools.partial(
    pl.kernel,
    mesh=_mesh,
    out_type=jax.ShapeDtypeStruct((BATCH // GROUP, GROUP, EMBED_DIM), jnp.float32),
    scratch_types=[
        pltpu.VMEM((B_PER_W,), jnp.int32),
        pltpu.VMEM((CHUNK, GROUP, EMBED_DIM), jnp.float32),
        pltpu.VMEM((CHUNK, GROUP, EMBED_DIM), jnp.float32),
        pltpu.VMEM((CHUNK // GROUP, GROUP, EMBED_DIM), jnp.float32),
        pltpu.VMEM((CHUNK // GROUP, GROUP, EMBED_DIM), jnp.float32),
        pltpu.SemaphoreType.DMA,
        pltpu.SemaphoreType.DMA,
        pltpu.SemaphoreType.DMA,
        pltpu.SemaphoreType.DMA,
    ],
    compiler_params=pltpu.CompilerParams(needs_layout_passes=False),
)
def _gather_kernel(
    table_hbm, idx_hbm, out_hbm, idx_v, slab0, slab1, outb0, outb1,
    gsem0, gsem1, wsem0, wsem1,
):
    wid = lax.axis_index("s") * NUM_CORES + lax.axis_index("c")
    base = wid * B_PER_W
    slabs = (slab0, slab1)
    outbs = (outb0, outb1)
    gsems = (gsem0, gsem1)
    wsems = (wsem0, wsem1)

    pltpu.sync_copy(idx_hbm.at[pl.ds(base, B_PER_W)], idx_v)

    def start_gather(j):
        slab = slabs[j % 2]
        sem = gsems[j % 2]
        for g in range(CHUNK // LANES):
            iv = idx_v[pl.ds(j * CHUNK + g * LANES, LANES)]
            sv = lax.shift_right_logical(iv, 3)
            for i in range(LANES):
                pltpu.async_copy(table_hbm.at[sv[i]], slab.at[g * LANES + i], sem)

    def drain_gather(j):
        pltpu.make_async_copy(
            table_hbm.at[pl.ds(0, CHUNK)], slabs[j % 2], gsems[j % 2]
        ).wait()

    def extract(j):
        slab = slabs[j % 2]
        outb = outbs[j % 2]
        iota = lax.iota(jnp.int32, LANES)
        for g in range(CHUNK // LANES):
            iv = idx_v[pl.ds(j * CHUNK + g * LANES, LANES)]
            rv = lax.bitwise_and(iv, 7)
            pv = iota + g * LANES
            pv_hi = lax.shift_right_logical(pv, 3)
            pv_lo = lax.bitwise_and(pv, 7)

            def body(c, _, rv=rv, pv=pv, pv_hi=pv_hi, pv_lo=pv_lo):
                cc = jnp.full((LANES,), c, jnp.int32)
                vals = plsc.load_gather(slab, [pv, rv, cc])
                plsc.store_scatter(outb, [pv_hi, pv_lo, cc], vals)
                return 0

            lax.fori_loop(0, EMBED_DIM, body, 0, unroll=4)

    def start_write(j):
        return pltpu.async_copy(
            outbs[j % 2],
            out_hbm.at[pl.ds(base // GROUP + j * (CHUNK // GROUP), CHUNK // GROUP)],
            wsems[j % 2],
        )

    writes = [None, None]
    start_gather(0)
    for j in range(N_CHUNKS):
        if j + 1 < N_CHUNKS:
            start_gather(j + 1)
        drain_gather(j)
        if writes[j % 2] is not None:
            writes[j % 2].wait()
        extract(j)
        writes[j % 2] = start_write(j)
    writes[(N_CHUNKS - 2) % 2].wait()
    writes[(N_CHUNKS - 1) % 2].wait()


def kernel(embed_cache, indices):
    table3 = embed_cache.reshape(VOCAB // GROUP, GROUP, EMBED_DIM)
    idx = indices.astype(jnp.int32)
    out3 = _gather_kernel(table3, idx)
    return out3.reshape(BATCH, EMBED_DIM)
